# baseline (speedup 1.0000x reference)
import functools

import jax
import jax.numpy as jnp
from jax import lax
from jax.experimental import pallas as pl
from jax.experimental.pallas import tpu as pltpu
from jax.experimental.pallas import tpu_sc as plsc

HIDDEN = 64
VOCAB = 2016
SCALER = 7

NC = 2
NS = 16
NW = NC * NS

B, T = 4096, 200
TOTAL = B * T
PER_W = TOTAL // NW
G0 = 128
FB_W = PER_W // G0
SBK = 2
NSB = FB_W // SBK


def _prep_body(emb_ref, w_ref, b_ref, tod_ref, dow_ref, table_ref, idx_ref):
    t = lax.dot_general(
        emb_ref[:], w_ref[:], (((1,), (1,)), ((), ())),
        preferred_element_type=jnp.float32,
    )
    t = t + b_ref[:]
    table_ref[:] = jnp.concatenate([t[:, 1:], jnp.sin(t[:, :1])], axis=1)
    idx_ref[:] = tod_ref[:] * SCALER + dow_ref[:]


def _prep(emb, W, b, tod, dow):
    return pl.pallas_call(
        _prep_body,
        out_shape=(
            jax.ShapeDtypeStruct((VOCAB, HIDDEN), jnp.float32),
            jax.ShapeDtypeStruct((B, T), jnp.int32),
        ),
    )(emb, W, b.reshape(1, HIDDEN), tod, dow)


@functools.partial(
    pl.kernel,
    mesh=plsc.VectorSubcoreMesh(core_axis_name="c", subcore_axis_name="s"),
    compiler_params=pltpu.CompilerParams(use_tc_tiling_on_sc=False),
    out_type=jax.ShapeDtypeStruct((TOTAL // G0, G0, HIDDEN), jnp.float32),
    scratch_types=[
        pltpu.VMEM((PER_W,), jnp.int32),
        pltpu.VMEM((3, SBK, G0, HIDDEN), jnp.float32),
        pltpu.VMEM_SHARED((VOCAB, HIDDEN), jnp.float32),
        pltpu.SemaphoreType.DMA,
        pltpu.SemaphoreType.DMA,
        pltpu.SemaphoreType.DMA,
        pltpu.SemaphoreType.DMA,
        pltpu.SemaphoreType.DMA,
        pltpu.SemaphoreType.DMA,
        pltpu.SemaphoreType.DMA,
    ],
)
def _sc_gather(idx_hbm, table_hbm, out_hbm,
               idx_v, rows_v, table_sh,
               ssem, gsem0, gsem1, gsem2, wsem0, wsem1, wsem2):
    sid = lax.axis_index("s")
    wid = sid * NC + lax.axis_index("c")
    base = wid * PER_W
    fb0 = wid * FB_W
    gsem = (gsem0, gsem1, gsem2)
    wsem = (wsem0, wsem1, wsem2)

    cp = pltpu.async_copy(idx_hbm.at[pl.ds(base, PER_W)], idx_v, ssem)

    @pl.when(sid == 0)
    def _():
        pltpu.sync_copy(table_hbm, table_sh)

    plsc.subcore_barrier()
    cp.wait()

    def start_gathers(sb, b):
        for k in range(SBK):
            toff = sb * SBK * G0 + k * G0
            pltpu.async_copy(
                table_sh.at[idx_v.at[pl.ds(toff, G0)]],
                rows_v.at[b, k], gsem[b])

    def wait_gathers(b):
        for k in range(SBK):
            pltpu.make_async_copy(
                table_sh.at[idx_v.at[pl.ds(0, G0)]],
                rows_v.at[b, k], gsem[b]).wait()

    def start_write(sb, b):
        pltpu.async_copy(rows_v.at[b],
                         out_hbm.at[pl.ds(fb0 + sb * SBK, SBK)], wsem[b])

    def wait_write(b):
        pltpu.make_async_copy(rows_v.at[b],
                              out_hbm.at[pl.ds(fb0, SBK)], wsem[b]).wait()

    start_gathers(0, 0)
    start_gathers(1, 1)

    def body(s, carry):
        for b in range(3):
            @pl.when(s % 3 == b)
            def _():
                wait_gathers(b)
                start_write(s, b)
                bn = (b + 2) % 3

                @pl.when(s + 2 < NSB)
                def _():
                    @pl.when(s >= 1)
                    def _():
                        wait_write(bn)
                    start_gathers(s + 2, bn)
        return carry

    lax.fori_loop(0, NSB, body, 0)
    wait_write(0)
    wait_write(1)
    wait_write(2)


def kernel(tod, dow, emb, W, b):
    table, idx = _prep(emb, W, b, tod, dow)
    out = _sc_gather(idx.reshape(-1), table)
    return out.reshape(B, T, HIDDEN)
